# baseline (speedup 1.0000x reference)
import jax
import jax.numpy as jnp
from jax.experimental import pallas as pl

_N = 4096
_D = 16
_BLK1 = 2048
_GRID1 = _N // _BLK1
_BLK = 1024
_GRID = _N // _BLK
_TGT_MED = 16
_TGT_THR = 31
_MAXBITS = 0x7F7FFFFF


def _d2_block(xb, xt, sqr, sqc):
    mm = jax.lax.dot_general(
        xb, xt, (((1,), (0,)), ((), ())),
        preferred_element_type=jnp.float32,
        precision=jax.lax.Precision.DEFAULT,
    )
    return jnp.maximum(sqr + sqc - 2.0 * mm, 0.0)


def _phase1(xb_ref, xt_ref, sqr_ref, sqc_ref, thr_ref, nsi_ref,
            thr_c_ref, nsi_c_ref):
    d2 = _d2_block(xb_ref[...], xt_ref[...], sqr_ref[...], sqc_ref[...])
    bits = jax.lax.bitcast_convert_type(d2, jnp.int32)
    r = d2.shape[0]
    lo0 = jnp.zeros((r, 1), jnp.int32)
    hi0 = jnp.full((r, 1), _MAXBITS, jnp.int32)

    def body_both(_, c):
        lo_m, hi_m, lo_t, hi_t = c
        mid_m = lo_m + ((hi_m - lo_m) >> 1)
        mid_t = lo_t + ((hi_t - lo_t) >> 1)
        cnt_m = jnp.sum((bits <= mid_m).astype(jnp.int32), axis=1, keepdims=True)
        cnt_t = jnp.sum((bits <= mid_t).astype(jnp.int32), axis=1, keepdims=True)
        ge_m = cnt_m >= _TGT_MED
        ge_t = cnt_t >= _TGT_THR
        return (
            jnp.where(ge_m, lo_m, mid_m + 1),
            jnp.where(ge_m, mid_m, hi_m),
            jnp.where(ge_t, lo_t, mid_t + 1),
            jnp.where(ge_t, mid_t, hi_t),
        )

    def body_thr(_, c):
        lo_t, hi_t = c
        mid_t = lo_t + ((hi_t - lo_t) >> 1)
        cnt_t = jnp.sum((bits <= mid_t).astype(jnp.int32), axis=1, keepdims=True)
        ge_t = cnt_t >= _TGT_THR
        return jnp.where(ge_t, lo_t, mid_t + 1), jnp.where(ge_t, mid_t, hi_t)

    lo_m, hi_m, lo_t, hi_t = jax.lax.fori_loop(
        0, 16, body_both, (lo0, hi0, lo0, hi0))
    lo_t, _ = jax.lax.fori_loop(0, 15, body_thr, (lo_t, hi_t))
    med = jax.lax.bitcast_convert_type(lo_m + ((hi_m - lo_m) >> 1), jnp.float32)
    scale2 = jnp.maximum(med + 1e-12, 1e-14)
    nsi = -1.0 / scale2
    thr = jax.lax.bitcast_convert_type(lo_t, jnp.float32)
    nsi_ref[...] = nsi
    thr_ref[...] = thr
    nsi_c_ref[...] = jnp.transpose(nsi, (1, 0))
    thr_c_ref[...] = jnp.transpose(thr, (1, 0))


def _phase2(xb_ref, xt_ref, sqr_ref, sqc_ref, thr_r_ref, nsi_r_ref,
            thr_c_ref, nsi_c_ref, out_ref):
    d2 = _d2_block(xb_ref[...], xt_ref[...], sqr_ref[...], sqc_ref[...])
    wa = jnp.where(d2 <= thr_r_ref[...], jnp.exp(d2 * nsi_r_ref[...]), 0.0)
    wb = jnp.where(d2 <= thr_c_ref[...], jnp.exp(d2 * nsi_c_ref[...]), 0.0)
    out_ref[...] = 0.5 * (wa + wb)


def kernel(X):
    X = X.astype(jnp.float32)
    sq = jnp.sum(X * X, axis=1)
    sqr = sq.reshape(_N, 1)
    sqc = sq.reshape(1, _N)
    xt = X.T

    thr_r, nsi_r, thr_c, nsi_c = pl.pallas_call(
        _phase1,
        grid=(_GRID1,),
        in_specs=[
            pl.BlockSpec((_BLK1, _D), lambda i: (i, 0)),
            pl.BlockSpec((_D, _N), lambda i: (0, 0)),
            pl.BlockSpec((_BLK1, 1), lambda i: (i, 0)),
            pl.BlockSpec((1, _N), lambda i: (0, 0)),
        ],
        out_specs=[
            pl.BlockSpec((_BLK1, 1), lambda i: (i, 0)),
            pl.BlockSpec((_BLK1, 1), lambda i: (i, 0)),
            pl.BlockSpec((1, _BLK1), lambda i: (0, i)),
            pl.BlockSpec((1, _BLK1), lambda i: (0, i)),
        ],
        out_shape=[
            jax.ShapeDtypeStruct((_N, 1), jnp.float32),
            jax.ShapeDtypeStruct((_N, 1), jnp.float32),
            jax.ShapeDtypeStruct((1, _N), jnp.float32),
            jax.ShapeDtypeStruct((1, _N), jnp.float32),
        ],
    )(X, xt, sqr, sqc)

    return pl.pallas_call(
        _phase2,
        grid=(_GRID,),
        in_specs=[
            pl.BlockSpec((_BLK, _D), lambda i: (i, 0)),
            pl.BlockSpec((_D, _N), lambda i: (0, 0)),
            pl.BlockSpec((_BLK, 1), lambda i: (i, 0)),
            pl.BlockSpec((1, _N), lambda i: (0, 0)),
            pl.BlockSpec((_BLK, 1), lambda i: (i, 0)),
            pl.BlockSpec((_BLK, 1), lambda i: (i, 0)),
            pl.BlockSpec((1, _N), lambda i: (0, 0)),
            pl.BlockSpec((1, _N), lambda i: (0, 0)),
        ],
        out_specs=pl.BlockSpec((_BLK, _N), lambda i: (i, 0)),
        out_shape=jax.ShapeDtypeStruct((_N, _N), jnp.float32),
    )(X, xt, sqr, sqc, thr_r, nsi_r, thr_c, nsi_c)


# speedup vs baseline: 1.0011x; 1.0011x over previous
import jax
import jax.numpy as jnp
from jax.experimental import pallas as pl

_N = 4096
_D = 16
_BLK1 = 2048
_GRID1 = _N // _BLK1
_BLK = 512
_GRID = _N // _BLK
_TGT_MED = 16
_TGT_THR = 31
_MAXBITS = 0x7F7FFFFF


def _d2_block(xb, xt, sqr, sqc):
    mm = jax.lax.dot_general(
        xb, xt, (((1,), (0,)), ((), ())),
        preferred_element_type=jnp.float32,
        precision=jax.lax.Precision.DEFAULT,
    )
    return jnp.maximum(sqr + sqc - 2.0 * mm, 0.0)


def _phase1(xb_ref, xt_ref, sqr_ref, sqc_ref, thr_ref, nsi_ref,
            thr_c_ref, nsi_c_ref):
    d2 = _d2_block(xb_ref[...], xt_ref[...], sqr_ref[...], sqc_ref[...])
    bits = jax.lax.bitcast_convert_type(d2, jnp.int32)
    r = d2.shape[0]
    lo0 = jnp.zeros((r, 1), jnp.int32)
    hi0 = jnp.full((r, 1), _MAXBITS, jnp.int32)

    def body_both(_, c):
        lo_m, hi_m, lo_t, hi_t = c
        mid_m = lo_m + ((hi_m - lo_m) >> 1)
        mid_t = lo_t + ((hi_t - lo_t) >> 1)
        cnt_m = jnp.sum((bits <= mid_m).astype(jnp.int32), axis=1, keepdims=True)
        cnt_t = jnp.sum((bits <= mid_t).astype(jnp.int32), axis=1, keepdims=True)
        ge_m = cnt_m >= _TGT_MED
        ge_t = cnt_t >= _TGT_THR
        return (
            jnp.where(ge_m, lo_m, mid_m + 1),
            jnp.where(ge_m, mid_m, hi_m),
            jnp.where(ge_t, lo_t, mid_t + 1),
            jnp.where(ge_t, mid_t, hi_t),
        )

    def body_thr(_, c):
        lo_t, hi_t = c
        mid_t = lo_t + ((hi_t - lo_t) >> 1)
        cnt_t = jnp.sum((bits <= mid_t).astype(jnp.int32), axis=1, keepdims=True)
        ge_t = cnt_t >= _TGT_THR
        return jnp.where(ge_t, lo_t, mid_t + 1), jnp.where(ge_t, mid_t, hi_t)

    lo_m, hi_m, lo_t, hi_t = jax.lax.fori_loop(
        0, 16, body_both, (lo0, hi0, lo0, hi0))
    lo_t, _ = jax.lax.fori_loop(0, 15, body_thr, (lo_t, hi_t))
    med = jax.lax.bitcast_convert_type(lo_m + ((hi_m - lo_m) >> 1), jnp.float32)
    scale2 = jnp.maximum(med + 1e-12, 1e-14)
    nsi = -1.0 / scale2
    thr = jax.lax.bitcast_convert_type(lo_t, jnp.float32)
    nsi_ref[...] = nsi
    thr_ref[...] = thr
    nsi_c_ref[...] = jnp.transpose(nsi, (1, 0))
    thr_c_ref[...] = jnp.transpose(thr, (1, 0))


def _phase2(xb_ref, xt_ref, sqr_ref, sqc_ref, thr_r_ref, nsi_r_ref,
            thr_c_ref, nsi_c_ref, out_ref):
    d2 = _d2_block(xb_ref[...], xt_ref[...], sqr_ref[...], sqc_ref[...])
    wa = jnp.where(d2 <= thr_r_ref[...], jnp.exp(d2 * nsi_r_ref[...]), 0.0)
    wb = jnp.where(d2 <= thr_c_ref[...], jnp.exp(d2 * nsi_c_ref[...]), 0.0)
    out_ref[...] = 0.5 * (wa + wb)


def kernel(X):
    X = X.astype(jnp.float32)
    sq = jnp.sum(X * X, axis=1)
    sqr = sq.reshape(_N, 1)
    sqc = sq.reshape(1, _N)
    xt = X.T

    thr_r, nsi_r, thr_c, nsi_c = pl.pallas_call(
        _phase1,
        grid=(_GRID1,),
        in_specs=[
            pl.BlockSpec((_BLK1, _D), lambda i: (i, 0)),
            pl.BlockSpec((_D, _N), lambda i: (0, 0)),
            pl.BlockSpec((_BLK1, 1), lambda i: (i, 0)),
            pl.BlockSpec((1, _N), lambda i: (0, 0)),
        ],
        out_specs=[
            pl.BlockSpec((_BLK1, 1), lambda i: (i, 0)),
            pl.BlockSpec((_BLK1, 1), lambda i: (i, 0)),
            pl.BlockSpec((1, _BLK1), lambda i: (0, i)),
            pl.BlockSpec((1, _BLK1), lambda i: (0, i)),
        ],
        out_shape=[
            jax.ShapeDtypeStruct((_N, 1), jnp.float32),
            jax.ShapeDtypeStruct((_N, 1), jnp.float32),
            jax.ShapeDtypeStruct((1, _N), jnp.float32),
            jax.ShapeDtypeStruct((1, _N), jnp.float32),
        ],
    )(X, xt, sqr, sqc)

    return pl.pallas_call(
        _phase2,
        grid=(_GRID,),
        in_specs=[
            pl.BlockSpec((_BLK, _D), lambda i: (i, 0)),
            pl.BlockSpec((_D, _N), lambda i: (0, 0)),
            pl.BlockSpec((_BLK, 1), lambda i: (i, 0)),
            pl.BlockSpec((1, _N), lambda i: (0, 0)),
            pl.BlockSpec((_BLK, 1), lambda i: (i, 0)),
            pl.BlockSpec((_BLK, 1), lambda i: (i, 0)),
            pl.BlockSpec((1, _N), lambda i: (0, 0)),
            pl.BlockSpec((1, _N), lambda i: (0, 0)),
        ],
        out_specs=pl.BlockSpec((_BLK, _N), lambda i: (i, 0)),
        out_shape=jax.ShapeDtypeStruct((_N, _N), jnp.float32),
    )(X, xt, sqr, sqc, thr_r, nsi_r, thr_c, nsi_c)


# speedup vs baseline: 1.0014x; 1.0003x over previous
import jax
import jax.numpy as jnp
from jax.experimental import pallas as pl

_N = 4096
_D = 16
_BLK1 = 2048
_GRID1 = _N // _BLK1
_BLK = 512
_GRID = _N // _BLK
_TGT_MED = 16
_TGT_THR = 31
_MAXBITS = 0x7F7FFFFF


def _d2_block(xb, xt, sqr, sqc):
    mm = jax.lax.dot_general(
        xb, xt, (((1,), (0,)), ((), ())),
        preferred_element_type=jnp.float32,
        precision=jax.lax.Precision.DEFAULT,
    )
    return jnp.maximum(sqr + sqc - 2.0 * mm, 0.0)


def _phase1(xb_ref, xt_ref, sqr_ref, sqc_ref, thr_ref, nsi_ref,
            thr_c_ref, nsi_c_ref):
    d2 = _d2_block(xb_ref[...], xt_ref[...], sqr_ref[...], sqc_ref[...])
    r = d2.shape[0]
    lo0 = jnp.zeros((r, 1), jnp.int32)
    hi0 = jnp.full((r, 1), _MAXBITS, jnp.int32)

    def body_both(_, c):
        lo_m, hi_m, lo_t, hi_t = c
        mid_m = lo_m + ((hi_m - lo_m) >> 1)
        mid_t = lo_t + ((hi_t - lo_t) >> 1)
        midf_m = jax.lax.bitcast_convert_type(mid_m, jnp.float32)
        midf_t = jax.lax.bitcast_convert_type(mid_t, jnp.float32)
        cnt_m = jnp.sum((d2 <= midf_m).astype(jnp.int32), axis=1, keepdims=True)
        cnt_t = jnp.sum((d2 <= midf_t).astype(jnp.int32), axis=1, keepdims=True)
        ge_m = cnt_m >= _TGT_MED
        ge_t = cnt_t >= _TGT_THR
        return (
            jnp.where(ge_m, lo_m, mid_m + 1),
            jnp.where(ge_m, mid_m, hi_m),
            jnp.where(ge_t, lo_t, mid_t + 1),
            jnp.where(ge_t, mid_t, hi_t),
        )

    def body_thr(_, c):
        lo_t, hi_t = c
        mid_t = lo_t + ((hi_t - lo_t) >> 1)
        midf_t = jax.lax.bitcast_convert_type(mid_t, jnp.float32)
        cnt_t = jnp.sum((d2 <= midf_t).astype(jnp.int32), axis=1, keepdims=True)
        ge_t = cnt_t >= _TGT_THR
        return jnp.where(ge_t, lo_t, mid_t + 1), jnp.where(ge_t, mid_t, hi_t)

    lo_m, hi_m, lo_t, hi_t = jax.lax.fori_loop(
        0, 16, body_both, (lo0, hi0, lo0, hi0))
    lo_t, _ = jax.lax.fori_loop(0, 15, body_thr, (lo_t, hi_t))
    med = jax.lax.bitcast_convert_type(lo_m + ((hi_m - lo_m) >> 1), jnp.float32)
    scale2 = jnp.maximum(med + 1e-12, 1e-14)
    nsi = -1.0 / scale2
    thr = jax.lax.bitcast_convert_type(lo_t, jnp.float32)
    nsi_ref[...] = nsi
    thr_ref[...] = thr
    nsi_c_ref[...] = jnp.transpose(nsi, (1, 0))
    thr_c_ref[...] = jnp.transpose(thr, (1, 0))


def _phase2(xb_ref, xt_ref, sqr_ref, sqc_ref, thr_r_ref, nsi_r_ref,
            thr_c_ref, nsi_c_ref, out_ref):
    d2 = _d2_block(xb_ref[...], xt_ref[...], sqr_ref[...], sqc_ref[...])
    wa = jnp.where(d2 <= thr_r_ref[...], jnp.exp(d2 * nsi_r_ref[...]), 0.0)
    wb = jnp.where(d2 <= thr_c_ref[...], jnp.exp(d2 * nsi_c_ref[...]), 0.0)
    out_ref[...] = 0.5 * (wa + wb)


def kernel(X):
    X = X.astype(jnp.float32)
    sq = jnp.sum(X * X, axis=1)
    sqr = sq.reshape(_N, 1)
    sqc = sq.reshape(1, _N)
    xt = X.T

    thr_r, nsi_r, thr_c, nsi_c = pl.pallas_call(
        _phase1,
        grid=(_GRID1,),
        in_specs=[
            pl.BlockSpec((_BLK1, _D), lambda i: (i, 0)),
            pl.BlockSpec((_D, _N), lambda i: (0, 0)),
            pl.BlockSpec((_BLK1, 1), lambda i: (i, 0)),
            pl.BlockSpec((1, _N), lambda i: (0, 0)),
        ],
        out_specs=[
            pl.BlockSpec((_BLK1, 1), lambda i: (i, 0)),
            pl.BlockSpec((_BLK1, 1), lambda i: (i, 0)),
            pl.BlockSpec((1, _BLK1), lambda i: (0, i)),
            pl.BlockSpec((1, _BLK1), lambda i: (0, i)),
        ],
        out_shape=[
            jax.ShapeDtypeStruct((_N, 1), jnp.float32),
            jax.ShapeDtypeStruct((_N, 1), jnp.float32),
            jax.ShapeDtypeStruct((1, _N), jnp.float32),
            jax.ShapeDtypeStruct((1, _N), jnp.float32),
        ],
    )(X, xt, sqr, sqc)

    return pl.pallas_call(
        _phase2,
        grid=(_GRID,),
        in_specs=[
            pl.BlockSpec((_BLK, _D), lambda i: (i, 0)),
            pl.BlockSpec((_D, _N), lambda i: (0, 0)),
            pl.BlockSpec((_BLK, 1), lambda i: (i, 0)),
            pl.BlockSpec((1, _N), lambda i: (0, 0)),
            pl.BlockSpec((_BLK, 1), lambda i: (i, 0)),
            pl.BlockSpec((_BLK, 1), lambda i: (i, 0)),
            pl.BlockSpec((1, _N), lambda i: (0, 0)),
            pl.BlockSpec((1, _N), lambda i: (0, 0)),
        ],
        out_specs=pl.BlockSpec((_BLK, _N), lambda i: (i, 0)),
        out_shape=jax.ShapeDtypeStruct((_N, _N), jnp.float32),
    )(X, xt, sqr, sqc, thr_r, nsi_r, thr_c, nsi_c)
